# baseline (speedup 1.0000x reference)
import functools

import jax
import jax.numpy as jnp
from jax import lax
from jax.experimental import pallas as pl
from jax.experimental.pallas import tpu as pltpu
from jax.experimental.pallas import tpu_sc as plsc

F32 = jnp.float32
I32 = jnp.int32

NW = 32
LN = 16
CH = 4096
BN = 256

N6, N5, N4, N3, N2 = 40962, 10242, 2562, 642, 162
NPAD = {N6: 41472, N5: 10752, N4: 3072, N3: 1024, N2: 512}


def _mesh():
    return plsc.VectorSubcoreMesh(
        core_axis_name="c", subcore_axis_name="s", num_cores=2, num_subcores=16
    )


def _wid():
    return lax.axis_index("s") * 2 + lax.axis_index("c")


@functools.lru_cache(None)
def _deg_kernel(epad, npd):
    n_per = npd // NW

    @functools.partial(
        pl.kernel,
        out_type=jax.ShapeDtypeStruct((npd,), F32),
        mesh=_mesh(),
        compiler_params=pltpu.CompilerParams(needs_layout_passes=False),
        scratch_types=[pltpu.VMEM((CH,), I32), pltpu.VMEM((n_per,), F32)],
    )
    def body(dst_hbm, out_hbm, dbuf, hist):
        n0 = _wid() * n_per

        def zero(i, _):
            hist[pl.ds(i * LN, LN)] = jnp.zeros((LN,), F32)
            return 0

        lax.fori_loop(0, n_per // LN, zero, 0)

        def chunk(k, _):
            pltpu.sync_copy(dst_hbm.at[pl.ds(k * CH, CH)], dbuf)

            def inner(i, _):
                d = dbuf[pl.ds(i * LN, LN)]
                m = (d >= n0) & (d < n0 + n_per)
                idx = jnp.where(m, d - n0, 0)
                v = jnp.where(m, jnp.full((LN,), 1.0, F32), jnp.zeros((LN,), F32))
                plsc.addupdate_scatter(hist, [idx], v)
                return 0

            lax.fori_loop(0, CH // LN, inner, 0)
            return 0

        lax.fori_loop(0, epad // CH, chunk, 0)

        def selfloop(i, _):
            sl = pl.ds(i * LN, LN)
            hist[sl] = hist[sl] + 1.0
            return 0

        lax.fori_loop(0, n_per // LN, selfloop, 0)
        pltpu.sync_copy(hist, out_hbm.at[pl.ds(n0, n_per)])

    return body


@functools.lru_cache(None)
def _norm_kernel(epad, npd):
    et = epad // NW

    @functools.partial(
        pl.kernel,
        out_type=jax.ShapeDtypeStruct((epad,), F32),
        mesh=_mesh(),
        compiler_params=pltpu.CompilerParams(needs_layout_passes=False),
        scratch_types=[
            pltpu.VMEM((et,), I32),
            pltpu.VMEM((et,), I32),
            pltpu.VMEM((et,), F32),
            pltpu.VMEM((npd,), F32),
        ],
    )
    def body(src_hbm, dst_hbm, dinv_hbm, out_hbm, sbuf, dbuf, nbuf, dcol):
        base = _wid() * et
        pltpu.sync_copy(src_hbm.at[pl.ds(base, et)], sbuf)
        pltpu.sync_copy(dst_hbm.at[pl.ds(base, et)], dbuf)
        pltpu.sync_copy(dinv_hbm, dcol)

        def inner(i, _):
            sl = pl.ds(i * LN, LN)
            a = plsc.load_gather(dcol, [sbuf[sl]])
            b = plsc.load_gather(dcol, [dbuf[sl]])
            nbuf[sl] = a * b
            return 0

        lax.fori_loop(0, et // LN, inner, 0)
        pltpu.sync_copy(nbuf, out_hbm.at[pl.ds(base, et)])

    return body


CT = 512


@functools.lru_cache(None)
def _agg_kernel(c, npd, epad, ch_sz, cpb, relu, npdst, res_mode):
    rounds = c // (NW * cpb)
    pooled = npdst > 0
    out_w = npdst if pooled else npd
    res_w = npdst if res_mode == 2 else npd

    scr_types = [pltpu.VMEM((npd,), F32) for _ in range(2 * cpb)]
    scr_types.append(pltpu.VMEM((3 * ch_sz,), I32))
    scr_types.append(pltpu.VMEM((max(c, LN),), F32))
    if res_mode:
        scr_types.append(pltpu.VMEM((res_w,), F32))
    if pooled:
        scr_types.append(pltpu.VMEM((npdst,), F32))
        scr_types.append(pltpu.VMEM((7 * CT,), I32))

    @functools.partial(
        pl.kernel,
        out_type=jax.ShapeDtypeStruct((c, out_w), F32),
        mesh=_mesh(),
        compiler_params=pltpu.CompilerParams(needs_layout_passes=False),
        scratch_types=scr_types,
    )
    def body(*args):
        n_in = 4 + (1 if res_mode else 0) + (1 if pooled else 0)
        xw_hbm, epk_hbm, dsq_hbm, b_hbm = args[:4]
        pos = 4
        res_hbm = hx_hbm = None
        if res_mode:
            res_hbm = args[pos]
            pos += 1
        if pooled:
            hx_hbm = args[pos]
            pos += 1
        out_hbm = args[pos]
        scr = args[pos + 1:]
        xcols = scr[:cpb]
        ocols = scr[cpb:2 * cpb]
        ebuf = scr[2 * cpb]
        bcol = scr[2 * cpb + 1]
        pos = 2 * cpb + 2
        rcol = pcol = hxb = None
        if res_mode:
            rcol = scr[pos]
            pos += 1
        if pooled:
            pcol = scr[pos]
            hxb = scr[pos + 1]

        wid = _wid()
        pltpu.sync_copy(b_hbm, bcol.at[pl.ds(0, c)])

        def round_(r, _):
            c0 = r * NW * cpb + wid * cpb
            for j in range(cpb):
                pltpu.sync_copy(xw_hbm.at[c0 + j], xcols[j])
                pltpu.sync_copy(dsq_hbm, ocols[j])

            def init(i, _):
                sl = pl.ds(i * LN, LN)
                for j in range(cpb):
                    ocols[j][sl] = ocols[j][sl] * xcols[j][sl]
                return 0

            lax.fori_loop(0, npd // LN, init, 0)

            def chunk(k, _):
                pltpu.sync_copy(epk_hbm.at[pl.ds(k * 3 * ch_sz, 3 * ch_sz)], ebuf)

                def inner(i, _):
                    s16 = ebuf[pl.ds(i * LN, LN)]
                    d16 = ebuf[pl.ds(ch_sz + i * LN, LN)]
                    nm = plsc.bitcast(ebuf[pl.ds(2 * ch_sz + i * LN, LN)], F32)
                    for j in range(cpb):
                        vals = plsc.load_gather(xcols[j], [s16]) * nm
                        plsc.addupdate_scatter(ocols[j], [d16], vals)
                    return 0

                lax.fori_loop(0, ch_sz // LN, inner, 0)
                return 0

            lax.fori_loop(0, epad // ch_sz, chunk, 0)

            for j in range(cpb):
                cj = c0 + j
                bj = plsc.load_gather(bcol, [jnp.zeros((LN,), I32) + cj])
                if res_mode:
                    pltpu.sync_copy(res_hbm.at[cj], rcol)
                if not pooled:
                    def fin(i, _):
                        sl = pl.ds(i * LN, LN)
                        v = ocols[j][sl] + bj
                        if res_mode == 1:
                            v = v + rcol[sl]
                        if relu:
                            v = jnp.maximum(v, 0.0)
                        ocols[j][sl] = v
                        return 0

                    lax.fori_loop(0, npd // LN, fin, 0)
                    pltpu.sync_copy(ocols[j], out_hbm.at[cj])
                else:
                    def pchunk(k2, _):
                        pltpu.sync_copy(
                            hx_hbm.at[pl.ds(k2 * 7 * CT, 7 * CT)], hxb)

                        def pin(i, _):
                            sl0 = i * LN
                            m = plsc.load_gather(
                                ocols[j], [hxb[pl.ds(sl0, LN)]])
                            for jj in range(1, 7):
                                g = plsc.load_gather(
                                    ocols[j], [hxb[pl.ds(jj * CT + sl0, LN)]])
                                m = jnp.maximum(m, g)
                            v = m + bj
                            if res_mode == 2:
                                v = v + rcol[pl.ds(k2 * CT + sl0, LN)]
                            if relu:
                                v = jnp.maximum(v, 0.0)
                            pcol[pl.ds(k2 * CT + sl0, LN)] = v
                            return 0

                        lax.fori_loop(0, CT // LN, pin, 0)
                        return 0

                    lax.fori_loop(0, npdst // CT, pchunk, 0)
                    pltpu.sync_copy(pcol, out_hbm.at[cj])
            return 0

        lax.fori_loop(0, rounds, round_, 0)

    return body


@functools.lru_cache(None)
def _pool_kernel(c, nps, npd):
    rounds = c // NW

    @functools.partial(
        pl.kernel,
        out_type=jax.ShapeDtypeStruct((c, npd), F32),
        mesh=_mesh(),
        compiler_params=pltpu.CompilerParams(needs_layout_passes=False),
        scratch_types=[
            pltpu.VMEM((nps,), F32),
            pltpu.VMEM((npd,), F32),
            pltpu.VMEM((7 * npd,), I32),
        ],
    )
    def body(x_hbm, hx_hbm, out_hbm, xcol, pcol, hxb):
        wid = _wid()
        pltpu.sync_copy(hx_hbm, hxb)

        def round_(r, _):
            ch = r * NW + wid
            pltpu.sync_copy(x_hbm.at[ch], xcol)

            def inner(i, _):
                sl0 = i * LN
                m = plsc.load_gather(xcol, [hxb[pl.ds(sl0, LN)]])
                for j in range(1, 7):
                    g = plsc.load_gather(xcol, [hxb[pl.ds(j * npd + sl0, LN)]])
                    m = jnp.maximum(m, g)
                pcol[pl.ds(sl0, LN)] = m
                return 0

            lax.fori_loop(0, npd // LN, inner, 0)
            pltpu.sync_copy(pcol, out_hbm.at[ch])
            return 0

        lax.fori_loop(0, rounds, round_, 0)

    return body


def _rsqrt_call(deg, n):
    npd = deg.shape[0]

    def body(d_ref, dinv_ref, dsq_ref):
        d = d_ref[...]
        col = lax.broadcasted_iota(I32, (1, npd), 1)
        dv = jnp.where(col < n, lax.rsqrt(d), 0.0)
        dinv_ref[...] = dv
        dsq_ref[...] = dv * dv

    dinv, dsq = pl.pallas_call(
        body,
        out_shape=(
            jax.ShapeDtypeStruct((1, npd), F32),
            jax.ShapeDtypeStruct((1, npd), F32),
        ),
    )(deg.reshape(1, npd))
    return dinv.reshape(npd), dsq.reshape(npd)


def _matmul_call(aT, w):
    k, npd = aT.shape
    cout = w.shape[1]

    def body(w_ref, a_ref, o_ref):
        o_ref[...] = lax.dot_general(
            w_ref[...], a_ref[...], (((0,), (0,)), ((), ())),
            preferred_element_type=F32,
        )

    return pl.pallas_call(
        body,
        grid=(npd // BN,),
        in_specs=[
            pl.BlockSpec((k, cout), lambda j: (0, 0)),
            pl.BlockSpec((k, BN), lambda j: (0, j)),
        ],
        out_specs=pl.BlockSpec((cout, BN), lambda j: (0, j)),
        out_shape=jax.ShapeDtypeStruct((cout, npd), F32),
    )(w, aT)


def _bias_act_call(aT, b, res=None, relu=True):
    c, npd = aT.shape
    b2 = b.reshape(c, 1)

    if res is None:
        def body(a_ref, b_ref, o_ref):
            v = a_ref[...] + b_ref[...]
            o_ref[...] = jnp.maximum(v, 0.0) if relu else v

        ins = [
            pl.BlockSpec((c, BN), lambda j: (0, j)),
            pl.BlockSpec((c, 1), lambda j: (0, 0)),
        ]
        args = (aT, b2)
    else:
        def body(a_ref, b_ref, r_ref, o_ref):
            v = a_ref[...] + b_ref[...] + r_ref[...]
            o_ref[...] = jnp.maximum(v, 0.0) if relu else v

        ins = [
            pl.BlockSpec((c, BN), lambda j: (0, j)),
            pl.BlockSpec((c, 1), lambda j: (0, 0)),
            pl.BlockSpec((c, BN), lambda j: (0, j)),
        ]
        args = (aT, b2, res)

    return pl.pallas_call(
        body,
        grid=(npd // BN,),
        in_specs=ins,
        out_specs=pl.BlockSpec((c, BN), lambda j: (0, j)),
        out_shape=jax.ShapeDtypeStruct((c, npd), F32),
    )(*args)


def _fc_call(hT, wT, b):
    c, npd = hT.shape

    def body(h_ref, w_ref, b_ref, o_ref):
        s = jnp.sum(h_ref[...] * w_ref[...]) + b_ref[0, 0]
        o_ref[...] = jnp.zeros((1, 1), F32) + s

    return pl.pallas_call(
        body,
        out_shape=jax.ShapeDtypeStruct((1, 1), F32),
    )(hT, wT, b.reshape(1, 1))


MAXCPB = {41472: 1, 10752: 4, 3072: 8, 1024: 16, 512: 16}


def _precompute(ei, n):
    npd = NPAD[n]
    ch_sz = 8192 if n == N6 else CH
    e = ei.shape[1]
    ep = ((e + ch_sz - 1) // ch_sz) * ch_sz
    src = jnp.pad(ei[0].astype(I32), (0, ep - e), constant_values=n)
    dst = jnp.pad(ei[1].astype(I32), (0, ep - e), constant_values=n)
    deg = _deg_kernel(ep, npd)(dst)
    dinv, dsq = _rsqrt_call(deg, n)
    nrm = _norm_kernel(ep, npd)(src, dst, dinv)
    nrmi = jax.lax.bitcast_convert_type(nrm, I32)
    epk = jnp.stack(
        [src.reshape(-1, ch_sz), dst.reshape(-1, ch_sz), nrmi.reshape(-1, ch_sz)],
        axis=1,
    ).reshape(-1)
    return {"epk": epk, "dsq": dsq, "np": npd, "ep": ep, "ch": ch_sz}


def _gcn(hT, w, pc, b, relu, res=None, npdst=0, hxpk=None):
    yT = _matmul_call(hT, w)
    c = w.shape[1]
    cpb = min(c // NW, MAXCPB[pc["np"]])
    res_mode = 0 if res is None else (2 if npdst else 1)
    args = [yT, pc["epk"], pc["dsq"], b]
    if res is not None:
        args.append(res)
    if npdst:
        args.append(hxpk)
    return _agg_kernel(c, pc["np"], pc["ep"], pc["ch"], cpb, relu, npdst,
                       res_mode)(*args)


def _hexpack(hx, l, npdst):
    hxp = jnp.pad(hx[:l].astype(I32), ((0, npdst - l), (0, 0)))
    return hxp.reshape(npdst // CT, CT, 7).transpose(0, 2, 1).reshape(-1)


def _impl(x, edge_index, e5, e4, e3, e2, hex6, hex5, hex4, hex3, params):
    pc6 = _precompute(edge_index, N6)
    pc55 = _precompute(e5, N5)
    pc45 = _precompute(e4, N5)
    pc44 = _precompute(e4, N4)
    pc33 = _precompute(e3, N3)
    pc22 = _precompute(e2, N2)

    x8 = jnp.zeros((8, NPAD[N6]), F32).at[:4, :N6].set(x.T)
    w0 = jnp.pad(params["w0"], ((0, 4), (0, 0)))
    h = _gcn(x8, w0, pc6, params["b0"], relu=True,
             npdst=NPAD[N5], hxpk=_hexpack(hex6, N5, NPAD[N5]))

    combos = [[pc55, pc45], [pc55, pc44], [pc44, pc33], [pc33, pc22]]
    pools = [
        None,
        (_hexpack(hex5, N4, NPAD[N4]), NPAD[N4]),
        (_hexpack(hex4, N3, NPAD[N3]), NPAD[N3]),
        (_hexpack(hex3, N2, NPAD[N2]), NPAD[N2]),
    ]

    for li, blks in enumerate(params["layers"]):
        for bi, p in enumerate(blks):
            pc = combos[li][bi]
            h1 = _gcn(h, p["w1"], pc, p["b1"], relu=True)
            if "dsw" in p:
                hf, npd = pools[li]
                p2 = _gcn(h1, p["w2"], pc, p["b2"], relu=False,
                          npdst=npd, hxpk=hf)
                h = _gcn(h, p["dsw"], pc, p["dsb"], relu=True, res=p2,
                         npdst=npd, hxpk=hf)
            else:
                h = _gcn(h1, p["w2"], pc, p["b2"], relu=True, res=h)

    wT = params["fc_w"].reshape(N2, 512).T
    wTp = jnp.zeros((512, NPAD[N2]), F32).at[:, :N2].set(wT)
    out = _fc_call(h, wTp, params["fc_b"])
    return out.reshape(1)


_run = jax.jit(_impl)


def kernel(x, edge_index, e5, e4, e3, e2, hex6, hex5, hex4, hex3, params):
    return _run(x, edge_index, e5, e4, e3, e2, hex6, hex5, hex4, hex3, params)


# speedup vs baseline: 1.0064x; 1.0064x over previous
import functools

import jax
import jax.numpy as jnp
from jax import lax
from jax.experimental import pallas as pl
from jax.experimental.pallas import tpu as pltpu
from jax.experimental.pallas import tpu_sc as plsc

F32 = jnp.float32
I32 = jnp.int32

NW = 32
LN = 16
CH = 4096
BN = 256

N6, N5, N4, N3, N2 = 40962, 10242, 2562, 642, 162
NPAD = {N6: 41472, N5: 10752, N4: 3072, N3: 1024, N2: 512}


def _mesh():
    return plsc.VectorSubcoreMesh(
        core_axis_name="c", subcore_axis_name="s", num_cores=2, num_subcores=16
    )


def _wid():
    return lax.axis_index("s") * 2 + lax.axis_index("c")


@functools.lru_cache(None)
def _deg_kernel(epad, npd):
    n_per = npd // NW

    @functools.partial(
        pl.kernel,
        out_type=jax.ShapeDtypeStruct((npd,), F32),
        mesh=_mesh(),
        compiler_params=pltpu.CompilerParams(needs_layout_passes=False),
        scratch_types=[pltpu.VMEM((CH,), I32), pltpu.VMEM((n_per,), F32)],
    )
    def body(dst_hbm, out_hbm, dbuf, hist):
        n0 = _wid() * n_per

        def zero(i, _):
            hist[pl.ds(i * LN, LN)] = jnp.zeros((LN,), F32)
            return 0

        lax.fori_loop(0, n_per // LN, zero, 0)

        def chunk(k, _):
            pltpu.sync_copy(dst_hbm.at[pl.ds(k * CH, CH)], dbuf)

            def inner(i, _):
                for u in range(4):
                    d = dbuf[pl.ds(i * LN * 4 + u * LN, LN)]
                    m = (d >= n0) & (d < n0 + n_per)
                    idx = jnp.where(m, d - n0, 0)
                    v = jnp.where(m, jnp.full((LN,), 1.0, F32),
                                  jnp.zeros((LN,), F32))
                    plsc.addupdate_scatter(hist, [idx], v)
                return 0

            lax.fori_loop(0, CH // (LN * 4), inner, 0)
            return 0

        lax.fori_loop(0, epad // CH, chunk, 0)

        def selfloop(i, _):
            sl = pl.ds(i * LN, LN)
            hist[sl] = hist[sl] + 1.0
            return 0

        lax.fori_loop(0, n_per // LN, selfloop, 0)
        pltpu.sync_copy(hist, out_hbm.at[pl.ds(n0, n_per)])

    return body


@functools.lru_cache(None)
def _norm_kernel(epad, npd):
    et = epad // NW

    @functools.partial(
        pl.kernel,
        out_type=jax.ShapeDtypeStruct((epad,), F32),
        mesh=_mesh(),
        compiler_params=pltpu.CompilerParams(needs_layout_passes=False),
        scratch_types=[
            pltpu.VMEM((et,), I32),
            pltpu.VMEM((et,), I32),
            pltpu.VMEM((et,), F32),
            pltpu.VMEM((npd,), F32),
        ],
    )
    def body(src_hbm, dst_hbm, dinv_hbm, out_hbm, sbuf, dbuf, nbuf, dcol):
        base = _wid() * et
        pltpu.sync_copy(src_hbm.at[pl.ds(base, et)], sbuf)
        pltpu.sync_copy(dst_hbm.at[pl.ds(base, et)], dbuf)
        pltpu.sync_copy(dinv_hbm, dcol)

        def inner(i, _):
            for u in range(4):
                sl = pl.ds(i * LN * 4 + u * LN, LN)
                a = plsc.load_gather(dcol, [sbuf[sl]])
                b = plsc.load_gather(dcol, [dbuf[sl]])
                nbuf[sl] = a * b
            return 0

        lax.fori_loop(0, et // (LN * 4), inner, 0)
        pltpu.sync_copy(nbuf, out_hbm.at[pl.ds(base, et)])

    return body


CT = 512


@functools.lru_cache(None)
def _agg_kernel(c, npd, epad, ch_sz, cpb, relu, npdst, res_mode):
    rounds = c // (NW * cpb)
    pooled = npdst > 0
    out_w = npdst if pooled else npd
    res_w = npdst if res_mode == 2 else npd

    scr_types = [pltpu.VMEM((npd,), F32) for _ in range(2 * cpb)]
    scr_types.append(pltpu.VMEM((3 * ch_sz,), I32))
    scr_types.append(pltpu.VMEM((max(c, LN),), F32))
    if res_mode:
        scr_types.append(pltpu.VMEM((res_w,), F32))
    if pooled:
        scr_types.append(pltpu.VMEM((npdst,), F32))
        scr_types.append(pltpu.VMEM((7 * CT,), I32))

    @functools.partial(
        pl.kernel,
        out_type=jax.ShapeDtypeStruct((c, out_w), F32),
        mesh=_mesh(),
        compiler_params=pltpu.CompilerParams(needs_layout_passes=False),
        scratch_types=scr_types,
    )
    def body(*args):
        n_in = 4 + (1 if res_mode else 0) + (1 if pooled else 0)
        xw_hbm, epk_hbm, dsq_hbm, b_hbm = args[:4]
        pos = 4
        res_hbm = hx_hbm = None
        if res_mode:
            res_hbm = args[pos]
            pos += 1
        if pooled:
            hx_hbm = args[pos]
            pos += 1
        out_hbm = args[pos]
        scr = args[pos + 1:]
        xcols = scr[:cpb]
        ocols = scr[cpb:2 * cpb]
        ebuf = scr[2 * cpb]
        bcol = scr[2 * cpb + 1]
        pos = 2 * cpb + 2
        rcol = pcol = hxb = None
        if res_mode:
            rcol = scr[pos]
            pos += 1
        if pooled:
            pcol = scr[pos]
            hxb = scr[pos + 1]

        wid = _wid()
        pltpu.sync_copy(b_hbm, bcol.at[pl.ds(0, c)])

        def round_(r, _):
            c0 = r * NW * cpb + wid * cpb
            for j in range(cpb):
                pltpu.sync_copy(xw_hbm.at[c0 + j], xcols[j])
                pltpu.sync_copy(dsq_hbm, ocols[j])

            def init(i, _):
                sl = pl.ds(i * LN, LN)
                for j in range(cpb):
                    ocols[j][sl] = ocols[j][sl] * xcols[j][sl]
                return 0

            lax.fori_loop(0, npd // LN, init, 0)

            def chunk(k, _):
                pltpu.sync_copy(epk_hbm.at[pl.ds(k * 3 * ch_sz, 3 * ch_sz)], ebuf)

                def inner(i, _):
                    for u in range(4):
                        off = i * LN * 4 + u * LN
                        s16 = ebuf[pl.ds(off, LN)]
                        d16 = ebuf[pl.ds(ch_sz + off, LN)]
                        nm = plsc.bitcast(ebuf[pl.ds(2 * ch_sz + off, LN)], F32)
                        for j in range(cpb):
                            vals = plsc.load_gather(xcols[j], [s16]) * nm
                            plsc.addupdate_scatter(ocols[j], [d16], vals)
                    return 0

                lax.fori_loop(0, ch_sz // (LN * 4), inner, 0)
                return 0

            lax.fori_loop(0, epad // ch_sz, chunk, 0)

            for j in range(cpb):
                cj = c0 + j
                bj = plsc.load_gather(bcol, [jnp.zeros((LN,), I32) + cj])
                if res_mode:
                    pltpu.sync_copy(res_hbm.at[cj], rcol)
                if not pooled:
                    def fin(i, _):
                        sl = pl.ds(i * LN, LN)
                        v = ocols[j][sl] + bj
                        if res_mode == 1:
                            v = v + rcol[sl]
                        if relu:
                            v = jnp.maximum(v, 0.0)
                        ocols[j][sl] = v
                        return 0

                    lax.fori_loop(0, npd // LN, fin, 0)
                    pltpu.sync_copy(ocols[j], out_hbm.at[cj])
                else:
                    def pchunk(k2, _):
                        pltpu.sync_copy(
                            hx_hbm.at[pl.ds(k2 * 7 * CT, 7 * CT)], hxb)

                        def pin(i, _):
                            sl0 = i * LN
                            m = plsc.load_gather(
                                ocols[j], [hxb[pl.ds(sl0, LN)]])
                            for jj in range(1, 7):
                                g = plsc.load_gather(
                                    ocols[j], [hxb[pl.ds(jj * CT + sl0, LN)]])
                                m = jnp.maximum(m, g)
                            v = m + bj
                            if res_mode == 2:
                                v = v + rcol[pl.ds(k2 * CT + sl0, LN)]
                            if relu:
                                v = jnp.maximum(v, 0.0)
                            pcol[pl.ds(k2 * CT + sl0, LN)] = v
                            return 0

                        lax.fori_loop(0, CT // LN, pin, 0)
                        return 0

                    lax.fori_loop(0, npdst // CT, pchunk, 0)
                    pltpu.sync_copy(pcol, out_hbm.at[cj])
            return 0

        lax.fori_loop(0, rounds, round_, 0)

    return body


@functools.lru_cache(None)
def _pool_kernel(c, nps, npd):
    rounds = c // NW

    @functools.partial(
        pl.kernel,
        out_type=jax.ShapeDtypeStruct((c, npd), F32),
        mesh=_mesh(),
        compiler_params=pltpu.CompilerParams(needs_layout_passes=False),
        scratch_types=[
            pltpu.VMEM((nps,), F32),
            pltpu.VMEM((npd,), F32),
            pltpu.VMEM((7 * npd,), I32),
        ],
    )
    def body(x_hbm, hx_hbm, out_hbm, xcol, pcol, hxb):
        wid = _wid()
        pltpu.sync_copy(hx_hbm, hxb)

        def round_(r, _):
            ch = r * NW + wid
            pltpu.sync_copy(x_hbm.at[ch], xcol)

            def inner(i, _):
                sl0 = i * LN
                m = plsc.load_gather(xcol, [hxb[pl.ds(sl0, LN)]])
                for j in range(1, 7):
                    g = plsc.load_gather(xcol, [hxb[pl.ds(j * npd + sl0, LN)]])
                    m = jnp.maximum(m, g)
                pcol[pl.ds(sl0, LN)] = m
                return 0

            lax.fori_loop(0, npd // LN, inner, 0)
            pltpu.sync_copy(pcol, out_hbm.at[ch])
            return 0

        lax.fori_loop(0, rounds, round_, 0)

    return body


def _rsqrt_call(deg, n):
    npd = deg.shape[0]

    def body(d_ref, dinv_ref, dsq_ref):
        d = d_ref[...]
        col = lax.broadcasted_iota(I32, (1, npd), 1)
        dv = jnp.where(col < n, lax.rsqrt(d), 0.0)
        dinv_ref[...] = dv
        dsq_ref[...] = dv * dv

    dinv, dsq = pl.pallas_call(
        body,
        out_shape=(
            jax.ShapeDtypeStruct((1, npd), F32),
            jax.ShapeDtypeStruct((1, npd), F32),
        ),
    )(deg.reshape(1, npd))
    return dinv.reshape(npd), dsq.reshape(npd)


def _matmul_call(aT, w):
    k, npd = aT.shape
    cout = w.shape[1]

    def body(w_ref, a_ref, o_ref):
        o_ref[...] = lax.dot_general(
            w_ref[...], a_ref[...], (((0,), (0,)), ((), ())),
            preferred_element_type=F32,
        )

    return pl.pallas_call(
        body,
        grid=(npd // BN,),
        in_specs=[
            pl.BlockSpec((k, cout), lambda j: (0, 0)),
            pl.BlockSpec((k, BN), lambda j: (0, j)),
        ],
        out_specs=pl.BlockSpec((cout, BN), lambda j: (0, j)),
        out_shape=jax.ShapeDtypeStruct((cout, npd), F32),
    )(w, aT)


def _bias_act_call(aT, b, res=None, relu=True):
    c, npd = aT.shape
    b2 = b.reshape(c, 1)

    if res is None:
        def body(a_ref, b_ref, o_ref):
            v = a_ref[...] + b_ref[...]
            o_ref[...] = jnp.maximum(v, 0.0) if relu else v

        ins = [
            pl.BlockSpec((c, BN), lambda j: (0, j)),
            pl.BlockSpec((c, 1), lambda j: (0, 0)),
        ]
        args = (aT, b2)
    else:
        def body(a_ref, b_ref, r_ref, o_ref):
            v = a_ref[...] + b_ref[...] + r_ref[...]
            o_ref[...] = jnp.maximum(v, 0.0) if relu else v

        ins = [
            pl.BlockSpec((c, BN), lambda j: (0, j)),
            pl.BlockSpec((c, 1), lambda j: (0, 0)),
            pl.BlockSpec((c, BN), lambda j: (0, j)),
        ]
        args = (aT, b2, res)

    return pl.pallas_call(
        body,
        grid=(npd // BN,),
        in_specs=ins,
        out_specs=pl.BlockSpec((c, BN), lambda j: (0, j)),
        out_shape=jax.ShapeDtypeStruct((c, npd), F32),
    )(*args)


def _fc_call(hT, wT, b):
    c, npd = hT.shape

    def body(h_ref, w_ref, b_ref, o_ref):
        s = jnp.sum(h_ref[...] * w_ref[...]) + b_ref[0, 0]
        o_ref[...] = jnp.zeros((1, 1), F32) + s

    return pl.pallas_call(
        body,
        out_shape=jax.ShapeDtypeStruct((1, 1), F32),
    )(hT, wT, b.reshape(1, 1))


MAXCPB = {41472: 1, 10752: 4, 3072: 8, 1024: 16, 512: 16}


def _precompute(ei, n):
    npd = NPAD[n]
    ch_sz = 8192 if n == N6 else CH
    e = ei.shape[1]
    ep = ((e + ch_sz - 1) // ch_sz) * ch_sz
    src = jnp.pad(ei[0].astype(I32), (0, ep - e), constant_values=n)
    dst = jnp.pad(ei[1].astype(I32), (0, ep - e), constant_values=n)
    deg = _deg_kernel(ep, npd)(dst)
    dinv, dsq = _rsqrt_call(deg, n)
    nrm = _norm_kernel(ep, npd)(src, dst, dinv)
    nrmi = jax.lax.bitcast_convert_type(nrm, I32)
    epk = jnp.stack(
        [src.reshape(-1, ch_sz), dst.reshape(-1, ch_sz), nrmi.reshape(-1, ch_sz)],
        axis=1,
    ).reshape(-1)
    return {"epk": epk, "dsq": dsq, "np": npd, "ep": ep, "ch": ch_sz}


def _gcn(hT, w, pc, b, relu, res=None, npdst=0, hxpk=None):
    yT = _matmul_call(hT, w)
    c = w.shape[1]
    cpb = min(c // NW, MAXCPB[pc["np"]])
    res_mode = 0 if res is None else (2 if npdst else 1)
    args = [yT, pc["epk"], pc["dsq"], b]
    if res is not None:
        args.append(res)
    if npdst:
        args.append(hxpk)
    return _agg_kernel(c, pc["np"], pc["ep"], pc["ch"], cpb, relu, npdst,
                       res_mode)(*args)


def _hexpack(hx, l, npdst):
    hxp = jnp.pad(hx[:l].astype(I32), ((0, npdst - l), (0, 0)))
    return hxp.reshape(npdst // CT, CT, 7).transpose(0, 2, 1).reshape(-1)


def _impl(x, edge_index, e5, e4, e3, e2, hex6, hex5, hex4, hex3, params):
    pc6 = _precompute(edge_index, N6)
    pc55 = _precompute(e5, N5)
    pc45 = _precompute(e4, N5)
    pc44 = _precompute(e4, N4)
    pc33 = _precompute(e3, N3)
    pc22 = _precompute(e2, N2)

    x8 = jnp.zeros((8, NPAD[N6]), F32).at[:4, :N6].set(x.T)
    w0 = jnp.pad(params["w0"], ((0, 4), (0, 0)))
    h = _gcn(x8, w0, pc6, params["b0"], relu=True,
             npdst=NPAD[N5], hxpk=_hexpack(hex6, N5, NPAD[N5]))

    combos = [[pc55, pc45], [pc55, pc44], [pc44, pc33], [pc33, pc22]]
    pools = [
        None,
        (_hexpack(hex5, N4, NPAD[N4]), NPAD[N4]),
        (_hexpack(hex4, N3, NPAD[N3]), NPAD[N3]),
        (_hexpack(hex3, N2, NPAD[N2]), NPAD[N2]),
    ]

    for li, blks in enumerate(params["layers"]):
        for bi, p in enumerate(blks):
            pc = combos[li][bi]
            h1 = _gcn(h, p["w1"], pc, p["b1"], relu=True)
            if "dsw" in p:
                hf, npd = pools[li]
                p2 = _gcn(h1, p["w2"], pc, p["b2"], relu=False,
                          npdst=npd, hxpk=hf)
                h = _gcn(h, p["dsw"], pc, p["dsb"], relu=True, res=p2,
                         npdst=npd, hxpk=hf)
            else:
                h = _gcn(h1, p["w2"], pc, p["b2"], relu=True, res=h)

    wT = params["fc_w"].reshape(N2, 512).T
    wTp = jnp.zeros((512, NPAD[N2]), F32).at[:, :N2].set(wT)
    out = _fc_call(h, wTp, params["fc_b"])
    return out.reshape(1)


_run = jax.jit(_impl)


def kernel(x, edge_index, e5, e4, e3, e2, hex6, hex5, hex4, hex3, params):
    return _run(x, edge_index, e5, e4, e3, e2, hex6, hex5, hex4, hex3, params)
